# baseline (speedup 1.0000x reference)
import functools

import jax
import jax.numpy as jnp
from jax import lax
from jax.experimental import pallas as pl
from jax.experimental.pallas import tpu as pltpu
from jax.experimental.pallas import tpu_sc as plsc

NC = 2
NS = 16
L = 16
NW = NC * NS
DC = 64


def _deg_call(N, E, B=400):
    EPW = E // NW
    NIT = EPW // B
    RPT = N // NS

    mesh = plsc.VectorSubcoreMesh(core_axis_name="c", subcore_axis_name="s")

    @functools.partial(
        pl.kernel,
        out_type=jax.ShapeDtypeStruct((NC, N, L), jnp.float32),
        mesh=mesh,
        scratch_types=[
            pltpu.VMEM((B,), jnp.int32),
            pltpu.VMEM((B,), jnp.int32),
            pltpu.VMEM((B, L), jnp.float32),
            pltpu.SemaphoreType.DMA,
            pltpu.SemaphoreType.DMA,
            pltpu.SemaphoreType.DMA,
            pltpu.SemaphoreType.DMA,
            pltpu.VMEM_SHARED((N, L), jnp.float32),
        ],
        compiler_params=pltpu.CompilerParams(use_tc_tiling_on_sc=False),
    )
    def deg_k(dst_hbm, zeros_hbm, out_hbm,
              dst_v0, dst_v1, ones_v, isem0, isem1, ssem0, ssem1, acc):
        cid = lax.axis_index("c")
        sid = lax.axis_index("s")
        wid = cid * NS + sid
        dst_vs = (dst_v0, dst_v1)
        isems = (isem0, isem1)
        ssems = (ssem0, ssem1)

        def fill(j, carry):
            ones_v[j, :] = jnp.full((L,), 1.0, jnp.float32)
            return carry

        lax.fori_loop(0, B, fill, 0)
        pltpu.sync_copy(zeros_hbm, acc.at[pl.ds(sid * RPT, RPT)])
        plsc.subcore_barrier()

        for b in range(2):
            pltpu.async_copy(dst_hbm.at[pl.ds(wid * EPW + b * B, B)],
                             dst_vs[b], isems[b])

        def step(k, carry):
            g = k * 2
            for b in range(2):
                i = g + b
                q = 1 - b

                @pl.when(i < NIT)
                def _():
                    pltpu.make_async_copy(
                        dst_hbm.at[pl.ds(0, B)], dst_vs[b], isems[b]).wait()
                    pltpu.async_copy(ones_v, acc.at[dst_vs[b]], ssems[b],
                                     add=True)

                    @pl.when((i >= 1) & (i + 1 < NIT))
                    def _():
                        pltpu.make_async_copy(
                            ones_v, acc.at[dst_vs[q]], ssems[q]).wait()
                        pltpu.async_copy(
                            dst_hbm.at[pl.ds(wid * EPW + (i + 1) * B, B)],
                            dst_vs[q], isems[q])

            return carry

        lax.fori_loop(0, (NIT + 1) // 2, step, 0)
        for b in range(2):
            pltpu.make_async_copy(ones_v, acc.at[dst_vs[b]], ssems[b]).wait()
        plsc.subcore_barrier()
        pltpu.sync_copy(acc.at[pl.ds(sid * RPT, RPT)],
                        out_hbm.at[cid, pl.ds(sid * RPT, RPT)])

    return deg_k


def _prop_call(N, E, D, B=400, NB=3):
    EPW = E // NW
    NIT = EPW // B
    RPT = N // NS
    P = D // DC

    mesh = plsc.VectorSubcoreMesh(core_axis_name="c", subcore_axis_name="s")

    @functools.partial(
        pl.kernel,
        out_type=jax.ShapeDtypeStruct((NC, N, D), jnp.float32),
        mesh=mesh,
        scratch_types=[
            [pltpu.VMEM((B,), jnp.int32) for _ in range(NB)],
            [pltpu.VMEM((B,), jnp.int32) for _ in range(NB)],
            [pltpu.VMEM((B, DC), jnp.float32) for _ in range(NB)],
            [pltpu.SemaphoreType.DMA for _ in range(NB)],
            [pltpu.SemaphoreType.DMA for _ in range(NB)],
            pltpu.VMEM_SHARED((N, DC), jnp.float32),
        ],
        compiler_params=pltpu.CompilerParams(use_tc_tiling_on_sc=False),
    )
    def prop_k(hs_hbm, src_hbm, dst_hbm, zeros_hbm, out_hbm,
               src_vs, dst_vs, rows_vs, gsems, ssems, acc):
        cid = lax.axis_index("c")
        sid = lax.axis_index("s")
        wid = cid * NS + sid

        for f in range(P):
            hs_f = hs_hbm.at[f]

            @pl.when(cid == 0)
            def _():
                pltpu.sync_copy(hs_f.at[pl.ds(sid * RPT, RPT)],
                                acc.at[pl.ds(sid * RPT, RPT)])

            @pl.when(cid != 0)
            def _():
                pltpu.sync_copy(zeros_hbm, acc.at[pl.ds(sid * RPT, RPT)])

            plsc.subcore_barrier()

            def fetch(j, b):
                base = wid * EPW + j * B
                pltpu.sync_copy(src_hbm.at[pl.ds(base, B)], src_vs[b])
                pltpu.sync_copy(dst_hbm.at[pl.ds(base, B)], dst_vs[b])
                pltpu.async_copy(hs_f.at[src_vs[b]], rows_vs[b], gsems[b])

            def wait_gather(b):
                pltpu.make_async_copy(hs_f.at[src_vs[b]],
                                      rows_vs[b], gsems[b]).wait()

            def wait_scatter(b):
                pltpu.make_async_copy(rows_vs[b], acc.at[dst_vs[b]],
                                      ssems[b]).wait()

            for b in range(NB):
                fetch(b, b)

            def step(k, carry):
                g = k * NB
                for b in range(NB):
                    i = g + b
                    q = (b + 2) % NB

                    @pl.when(i < NIT)
                    def _():
                        wait_gather(b)
                        pltpu.async_copy(rows_vs[b], acc.at[dst_vs[b]],
                                         ssems[b], add=True)

                        @pl.when((i + 2 >= NB) & (i + 2 < NIT))
                        def _():
                            wait_scatter(q)
                            fetch(i + 2, q)

                return carry

            lax.fori_loop(0, (NIT + NB - 1) // NB, step, 0)
            for b in range(NB):
                wait_scatter(b)
            plsc.subcore_barrier()
            pltpu.sync_copy(
                acc.at[pl.ds(sid * RPT, RPT)],
                out_hbm.at[cid, pl.ds(sid * RPT, RPT), pl.ds(f * DC, DC)])
            plsc.subcore_barrier()

    return prop_k


def _dinv_from_deg(d_ref):
    s = d_ref[0] + d_ref[1] + 1.0
    return lax.rsqrt(s)[:, 0:1]


def _mm1_call(X, Wpm, bpm):
    N, Din = X.shape
    P = Wpm.shape[0]
    R = 1000

    def body(x_ref, w_ref, b_ref, o_ref):
        o_ref[0] = jnp.dot(x_ref[...], w_ref[0],
                           preferred_element_type=jnp.float32) + b_ref[0]

    return pl.pallas_call(
        body,
        grid=(N // R, P),
        in_specs=[
            pl.BlockSpec((R, Din), lambda i, f: (i, 0)),
            pl.BlockSpec((1, Din, DC), lambda i, f: (f, 0, 0)),
            pl.BlockSpec((1, 1, DC), lambda i, f: (f, 0, 0)),
        ],
        out_specs=pl.BlockSpec((1, R, DC), lambda i, f: (f, i, 0)),
        out_shape=jax.ShapeDtypeStruct((P, N, DC), jnp.float32),
    )(X, Wpm, bpm)


def _scale_call(degp, hraw):
    P, N, _ = hraw.shape
    R = 1000

    def body(d_ref, h_ref, o_ref):
        dv = _dinv_from_deg(d_ref)
        o_ref[0] = dv * h_ref[0]

    return pl.pallas_call(
        body,
        grid=(N // R, P),
        in_specs=[
            pl.BlockSpec((NC, R, L), lambda i, f: (0, i, 0)),
            pl.BlockSpec((1, R, DC), lambda i, f: (f, i, 0)),
        ],
        out_specs=pl.BlockSpec((1, R, DC), lambda i, f: (f, i, 0)),
        out_shape=jax.ShapeDtypeStruct((P, N, DC), jnp.float32),
    )(degp, hraw)


def _mm2_call(degp, parts, Wpm, bpm):
    _, N, Din = parts.shape
    P = Wpm.shape[0]
    R = 1000

    def body(d_ref, p_ref, w_ref, b_ref, o_ref):
        dv = _dinv_from_deg(d_ref)
        act = jnp.maximum(dv * (p_ref[0] + p_ref[1]), 0.0)
        h = jnp.dot(act, w_ref[0],
                    preferred_element_type=jnp.float32) + b_ref[0]
        o_ref[0] = dv * h

    return pl.pallas_call(
        body,
        grid=(N // R, P),
        in_specs=[
            pl.BlockSpec((NC, R, L), lambda i, f: (0, i, 0)),
            pl.BlockSpec((NC, R, Din), lambda i, f: (0, i, 0)),
            pl.BlockSpec((1, Din, DC), lambda i, f: (f, 0, 0)),
            pl.BlockSpec((1, 1, DC), lambda i, f: (f, 0, 0)),
        ],
        out_specs=pl.BlockSpec((1, R, DC), lambda i, f: (f, i, 0)),
        out_shape=jax.ShapeDtypeStruct((P, N, DC), jnp.float32),
    )(degp, parts, Wpm, bpm)


def _final_call(degp, parts):
    _, N, D = parts.shape
    R = 1000

    def body(d_ref, p_ref, o_ref):
        dv = _dinv_from_deg(d_ref)
        o_ref[...] = dv * (p_ref[0] + p_ref[1])

    return pl.pallas_call(
        body,
        grid=(N // R,),
        in_specs=[
            pl.BlockSpec((NC, R, L), lambda i: (0, i, 0)),
            pl.BlockSpec((NC, R, D), lambda i: (0, i, 0)),
        ],
        out_specs=pl.BlockSpec((R, D), lambda i: (i, 0)),
        out_shape=jax.ShapeDtypeStruct((N, D), jnp.float32),
    )(degp, parts)


def kernel(X, edge_index, W1, b1, W2, b2):
    N, D1 = X.shape
    D2 = W2.shape[1]
    E = edge_index.shape[1]
    src = edge_index[0]
    dst = edge_index[1]

    RPT = N // NS
    zeros_deg = jnp.zeros((RPT, L), jnp.float32)
    zeros_dc = jnp.zeros((RPT, DC), jnp.float32)

    P1 = D1 // DC
    P2 = D2 // DC
    W1pm = W1.reshape(D1, P1, DC).transpose(1, 0, 2)
    b1pm = b1.reshape(P1, 1, DC)
    W2pm = W2.reshape(D1, P2, DC).transpose(1, 0, 2)
    b2pm = b2.reshape(P2, 1, DC)

    degp = _deg_call(N, E)(dst, zeros_deg)
    h1raw = _mm1_call(X, W1pm, b1pm)
    hs1 = _scale_call(degp, h1raw)
    p1 = _prop_call(N, E, D1)(hs1, src, dst, zeros_dc)
    hs2 = _mm2_call(degp, p1, W2pm, b2pm)
    p2 = _prop_call(N, E, D2)(hs2, src, dst, zeros_dc)
    return _final_call(degp, p2)


# speedup vs baseline: 1.0101x; 1.0101x over previous
import functools

import jax
import jax.numpy as jnp
from jax import lax
from jax.experimental import pallas as pl
from jax.experimental.pallas import tpu as pltpu
from jax.experimental.pallas import tpu_sc as plsc

NC = 2
NS = 16
L = 16
NW = NC * NS
DC = 64


def _deg_call(N, E, B=1000):
    EPW = E // NW
    NIT = EPW // B
    RPT = N // NS

    mesh = plsc.VectorSubcoreMesh(core_axis_name="c", subcore_axis_name="s")

    @functools.partial(
        pl.kernel,
        out_type=jax.ShapeDtypeStruct((NC, N, L), jnp.float32),
        mesh=mesh,
        scratch_types=[
            pltpu.VMEM((B,), jnp.int32),
            pltpu.VMEM((B,), jnp.int32),
            pltpu.VMEM((B, L), jnp.float32),
            pltpu.SemaphoreType.DMA,
            pltpu.SemaphoreType.DMA,
            pltpu.SemaphoreType.DMA,
            pltpu.SemaphoreType.DMA,
            pltpu.VMEM_SHARED((N, L), jnp.float32),
        ],
        compiler_params=pltpu.CompilerParams(use_tc_tiling_on_sc=False),
    )
    def deg_k(dst_hbm, zeros_hbm, out_hbm,
              dst_v0, dst_v1, ones_v, isem0, isem1, ssem0, ssem1, acc):
        cid = lax.axis_index("c")
        sid = lax.axis_index("s")
        wid = cid * NS + sid
        dst_vs = (dst_v0, dst_v1)
        isems = (isem0, isem1)
        ssems = (ssem0, ssem1)

        def fill(j, carry):
            ones_v[j, :] = jnp.full((L,), 1.0, jnp.float32)
            return carry

        lax.fori_loop(0, B, fill, 0)
        pltpu.sync_copy(zeros_hbm, acc.at[pl.ds(sid * RPT, RPT)])
        plsc.subcore_barrier()

        for b in range(2):
            pltpu.async_copy(dst_hbm.at[pl.ds(wid * EPW + b * B, B)],
                             dst_vs[b], isems[b])

        def step(k, carry):
            g = k * 2
            for b in range(2):
                i = g + b
                q = 1 - b

                @pl.when(i < NIT)
                def _():
                    pltpu.make_async_copy(
                        dst_hbm.at[pl.ds(0, B)], dst_vs[b], isems[b]).wait()
                    pltpu.async_copy(ones_v, acc.at[dst_vs[b]], ssems[b],
                                     add=True)

                    @pl.when((i >= 1) & (i + 1 < NIT))
                    def _():
                        pltpu.make_async_copy(
                            ones_v, acc.at[dst_vs[q]], ssems[q]).wait()
                        pltpu.async_copy(
                            dst_hbm.at[pl.ds(wid * EPW + (i + 1) * B, B)],
                            dst_vs[q], isems[q])

            return carry

        lax.fori_loop(0, (NIT + 1) // 2, step, 0)
        for b in range(2):
            pltpu.make_async_copy(ones_v, acc.at[dst_vs[b]], ssems[b]).wait()
        plsc.subcore_barrier()
        pltpu.sync_copy(acc.at[pl.ds(sid * RPT, RPT)],
                        out_hbm.at[cid, pl.ds(sid * RPT, RPT)])

    return deg_k


def _prop_call(N, E, D, B=400, NB=3):
    EPW = E // NW
    NIT = EPW // B
    RPT = N // NS
    P = D // DC

    mesh = plsc.VectorSubcoreMesh(core_axis_name="c", subcore_axis_name="s")

    @functools.partial(
        pl.kernel,
        out_type=jax.ShapeDtypeStruct((NC, N, D), jnp.float32),
        mesh=mesh,
        scratch_types=[
            [pltpu.VMEM((B,), jnp.int32) for _ in range(NB)],
            [pltpu.VMEM((B,), jnp.int32) for _ in range(NB)],
            [pltpu.VMEM((B, DC), jnp.float32) for _ in range(NB)],
            [pltpu.SemaphoreType.DMA for _ in range(NB)],
            [pltpu.SemaphoreType.DMA for _ in range(NB)],
            pltpu.VMEM_SHARED((N, DC), jnp.float32),
        ],
        compiler_params=pltpu.CompilerParams(use_tc_tiling_on_sc=False),
    )
    def prop_k(hs_hbm, src_hbm, dst_hbm, zeros_hbm, out_hbm,
               src_vs, dst_vs, rows_vs, gsems, ssems, acc):
        cid = lax.axis_index("c")
        sid = lax.axis_index("s")
        wid = cid * NS + sid

        for f in range(P):
            hs_f = hs_hbm.at[f]

            @pl.when(cid == 0)
            def _():
                pltpu.sync_copy(hs_f.at[pl.ds(sid * RPT, RPT)],
                                acc.at[pl.ds(sid * RPT, RPT)])

            @pl.when(cid != 0)
            def _():
                pltpu.sync_copy(zeros_hbm, acc.at[pl.ds(sid * RPT, RPT)])

            plsc.subcore_barrier()

            def fetch(j, b):
                base = wid * EPW + j * B
                pltpu.sync_copy(src_hbm.at[pl.ds(base, B)], src_vs[b])
                pltpu.sync_copy(dst_hbm.at[pl.ds(base, B)], dst_vs[b])
                pltpu.async_copy(hs_f.at[src_vs[b]], rows_vs[b], gsems[b])

            def wait_gather(b):
                pltpu.make_async_copy(hs_f.at[src_vs[b]],
                                      rows_vs[b], gsems[b]).wait()

            def wait_scatter(b):
                pltpu.make_async_copy(rows_vs[b], acc.at[dst_vs[b]],
                                      ssems[b]).wait()

            for b in range(NB):
                fetch(b, b)

            def step(k, carry):
                g = k * NB
                for b in range(NB):
                    i = g + b
                    q = (b + 2) % NB

                    @pl.when(i < NIT)
                    def _():
                        wait_gather(b)
                        pltpu.async_copy(rows_vs[b], acc.at[dst_vs[b]],
                                         ssems[b], add=True)

                        @pl.when((i + 2 >= NB) & (i + 2 < NIT))
                        def _():
                            wait_scatter(q)
                            fetch(i + 2, q)

                return carry

            lax.fori_loop(0, (NIT + NB - 1) // NB, step, 0)
            for b in range(NB):
                wait_scatter(b)
            plsc.subcore_barrier()
            pltpu.sync_copy(
                acc.at[pl.ds(sid * RPT, RPT)],
                out_hbm.at[cid, pl.ds(sid * RPT, RPT), pl.ds(f * DC, DC)])
            plsc.subcore_barrier()

    return prop_k


def _dinv_from_deg(d_ref):
    s = d_ref[0] + d_ref[1] + 1.0
    return lax.rsqrt(s)[:, 0:1]


def _mm1_call(degp, X, Wpm, bpm):
    N, Din = X.shape
    P = Wpm.shape[0]
    R = 1000

    def body(d_ref, x_ref, w_ref, b_ref, o_ref):
        dv = _dinv_from_deg(d_ref)
        h = jnp.dot(x_ref[...], w_ref[0],
                    preferred_element_type=jnp.float32) + b_ref[0]
        o_ref[0] = dv * h

    return pl.pallas_call(
        body,
        grid=(N // R, P),
        in_specs=[
            pl.BlockSpec((NC, R, L), lambda i, f: (0, i, 0)),
            pl.BlockSpec((R, Din), lambda i, f: (i, 0)),
            pl.BlockSpec((1, Din, DC), lambda i, f: (f, 0, 0)),
            pl.BlockSpec((1, 1, DC), lambda i, f: (f, 0, 0)),
        ],
        out_specs=pl.BlockSpec((1, R, DC), lambda i, f: (f, i, 0)),
        out_shape=jax.ShapeDtypeStruct((P, N, DC), jnp.float32),
    )(degp, X, Wpm, bpm)


def _mm2_call(degp, parts, Wpm, bpm):
    _, N, Din = parts.shape
    P = Wpm.shape[0]
    R = 1000

    def body(d_ref, p_ref, w_ref, b_ref, o_ref):
        dv = _dinv_from_deg(d_ref)
        act = jnp.maximum(dv * (p_ref[0] + p_ref[1]), 0.0)
        h = jnp.dot(act, w_ref[0],
                    preferred_element_type=jnp.float32) + b_ref[0]
        o_ref[0] = dv * h

    return pl.pallas_call(
        body,
        grid=(N // R, P),
        in_specs=[
            pl.BlockSpec((NC, R, L), lambda i, f: (0, i, 0)),
            pl.BlockSpec((NC, R, Din), lambda i, f: (0, i, 0)),
            pl.BlockSpec((1, Din, DC), lambda i, f: (f, 0, 0)),
            pl.BlockSpec((1, 1, DC), lambda i, f: (f, 0, 0)),
        ],
        out_specs=pl.BlockSpec((1, R, DC), lambda i, f: (f, i, 0)),
        out_shape=jax.ShapeDtypeStruct((P, N, DC), jnp.float32),
    )(degp, parts, Wpm, bpm)


def _final_call(degp, parts):
    _, N, D = parts.shape
    R = 1000

    def body(d_ref, p_ref, o_ref):
        dv = _dinv_from_deg(d_ref)
        o_ref[...] = dv * (p_ref[0] + p_ref[1])

    return pl.pallas_call(
        body,
        grid=(N // R,),
        in_specs=[
            pl.BlockSpec((NC, R, L), lambda i: (0, i, 0)),
            pl.BlockSpec((NC, R, D), lambda i: (0, i, 0)),
        ],
        out_specs=pl.BlockSpec((R, D), lambda i: (i, 0)),
        out_shape=jax.ShapeDtypeStruct((N, D), jnp.float32),
    )(degp, parts)


def kernel(X, edge_index, W1, b1, W2, b2):
    N, D1 = X.shape
    D2 = W2.shape[1]
    E = edge_index.shape[1]
    src = edge_index[0]
    dst = edge_index[1]

    RPT = N // NS
    zeros_deg = jnp.zeros((RPT, L), jnp.float32)
    zeros_dc = jnp.zeros((RPT, DC), jnp.float32)

    P1 = D1 // DC
    P2 = D2 // DC
    W1pm = W1.reshape(D1, P1, DC).transpose(1, 0, 2)
    b1pm = b1.reshape(P1, 1, DC)
    W2pm = W2.reshape(D1, P2, DC).transpose(1, 0, 2)
    b2pm = b2.reshape(P2, 1, DC)

    degp = _deg_call(N, E)(dst, zeros_deg)
    hs1 = _mm1_call(degp, X, W1pm, b1pm)
    p1 = _prop_call(N, E, D1)(hs1, src, dst, zeros_dc)
    hs2 = _mm2_call(degp, p1, W2pm, b2pm)
    p2 = _prop_call(N, E, D2)(hs2, src, dst, zeros_dc)
    return _final_call(degp, p2)
